# baseline (speedup 1.0000x reference)
import jax
import jax.numpy as jnp
from jax import lax
from jax.experimental import pallas as pl
from jax.experimental.pallas import tpu as pltpu
from jax.experimental.pallas import tpu_sc as plsc

N = 100000
E = 3200000
NB = 100
BN = N // NB
EPAD = 11264
EROWS = (E + EPAD) // 128
ROWS_PER_TILE = EROWS // 16
CHUNK_ROWS = 2
CHUNKS = ROWS_PER_TILE // CHUNK_ROWS
NPAD = 100096
ZROWS = NPAD // 16
FROWS = N // 16
VAR_EPS = 1e-4


def _enc_body(x_ref, w0_ref, b0_ref, w1_ref, b1_ref, wm_ref, bm_ref,
              wv_ref, bv_ref, wlm_ref, blm_ref, wrm_ref, brm_ref,
              wlv_ref, blv_ref, wrv_ref, brv_ref, atta_ref, eps_ref,
              p_ref, bnd_ref):
    i = pl.program_id(0)
    x = x_ref[...]
    lib = jnp.log(jnp.sum(x, axis=1, keepdims=True))
    xo = jnp.log1p(x)
    h = jax.nn.relu(jnp.dot(xo, w0_ref[...], preferred_element_type=jnp.float32) + b0_ref[...])
    h = jax.nn.relu(jnp.dot(h, w1_ref[...], preferred_element_type=jnp.float32) + b1_ref[...])
    qm = jnp.dot(h, wm_ref[...], preferred_element_type=jnp.float32) + bm_ref[...]
    qv = jnp.exp(jnp.dot(h, wv_ref[...], preferred_element_type=jnp.float32) + bv_ref[...]) + VAR_EPS
    z = qm + jnp.sqrt(qv) * eps_ref[...]
    qs = qm[:, 10:20]
    xlm = jnp.dot(qs, wlm_ref[...], preferred_element_type=jnp.float32) + blm_ref[...]
    xrm = jnp.dot(qs, wrm_ref[...], preferred_element_type=jnp.float32) + brm_ref[...]
    xlv = jnp.dot(qs, wlv_ref[...], preferred_element_type=jnp.float32) + blv_ref[...]
    xrv = jnp.dot(qs, wrv_ref[...], preferred_element_type=jnp.float32) + brv_ref[...]
    one = jnp.ones((BN, 1), jnp.float32)
    zero5 = jnp.zeros((BN, 5), jnp.float32)
    zero6 = jnp.zeros((BN, 6), jnp.float32)
    pad23 = jnp.zeros((BN, 23), jnp.float32)
    p_ref[...] = jnp.concatenate(
        [z, qm, lib, pad23,
         xlm, one, zero5, xrm, zero6,
         xlv, one, zero5, xrv, zero6], axis=1)
    am = atta_ref[0:1, 0:10]
    av = atta_ref[1:2, 0:10]
    pm = jnp.max(jnp.sum(jnp.abs(xlm) * am, axis=1))
    qmx = jnp.max(jnp.sum(jnp.abs(xrm) * am, axis=1))
    pv = jnp.max(jnp.sum(jnp.abs(xlv) * av, axis=1))
    qvx = jnp.max(jnp.sum(jnp.abs(xrv) * av, axis=1))
    lane = lax.broadcasted_iota(jnp.int32, (1, 128), 1)
    row = (jnp.where(lane == 0, pm, 0.0) + jnp.where(lane == 1, qmx, 0.0)
           + jnp.where(lane == 2, pv, 0.0) + jnp.where(lane == 3, qvx, 0.0))

    @pl.when(i == 0)
    def _():
        bnd_ref[...] = row

    @pl.when(i > 0)
    def _():
        bnd_ref[...] = jnp.maximum(bnd_ref[...], row)


def _encoder(x, w0t, b0, w1t, b1, wmt, bm, wvt, bv,
             wlmt, blm, wrmt, brm, wlvt, blv, wrvt, brv, atta, eps_z):
    full = lambda shape: pl.BlockSpec(shape, lambda i: (0, 0))
    blk = lambda shape: pl.BlockSpec(shape, lambda i: (i, 0))
    return pl.pallas_call(
        _enc_body,
        grid=(NB,),
        in_specs=[blk((BN, 128)), full((128, 128)), full((1, 128)),
                  full((128, 128)), full((1, 128)),
                  full((128, 20)), full((1, 20)), full((128, 20)), full((1, 20)),
                  full((10, 10)), full((1, 10)), full((10, 10)), full((1, 10)),
                  full((10, 10)), full((1, 10)), full((10, 10)), full((1, 10)),
                  full((2, 16)), blk((BN, 20))],
        out_specs=[blk((BN, 128)), full((1, 128))],
        out_shape=[jax.ShapeDtypeStruct((N, 128), jnp.float32),
                   jax.ShapeDtypeStruct((1, 128), jnp.float32)],
    )(x, w0t, b0, w1t, b1, wmt, bm, wvt, bv,
      wlmt, blm, wrmt, brm, wlvt, blv, wrvt, brv, atta, eps_z)


def _gat_sc_body(ei3, p_hbm, att2, cv2, acc_out,
                 accum, t0l, t0r, t1l, t1r,
                 sb0, db0, sb1, db1, sb2, db2, sb3, db3,
                 xl0, xr0, xl1, xr1, attb, cvb,
                 sem_g, sem_i0, sem_i1, sem_i2, sem_i3):
    c = lax.axis_index("c")
    s = lax.axis_index("s")
    idxbufs = [(sb0, db0), (sb1, db1), (sb2, db2), (sb3, db3)]
    rowbufs = [(xl0, xr0), (xl1, xr1)]
    sem_is = [sem_i0, sem_i1, sem_i2, sem_i3]

    pltpu.sync_copy(att2.at[c], attb)
    pltpu.sync_copy(cv2.at[c], cvb)
    cv = cvb[...]
    attv = attb[...]
    atts = [attv[k] for k in range(10)]
    cols = [jnp.full((16,), k, jnp.int32) for k in range(11)]

    fb = s * FROWS

    @pl.when(c == 0)
    def _():
        pltpu.sync_copy(p_hbm.at[pl.ds(fb, FROWS), pl.ds(64, 16)],
                        t0l.at[pl.ds(fb, FROWS)])
        pltpu.sync_copy(p_hbm.at[pl.ds(fb, FROWS), pl.ds(80, 16)],
                        t0r.at[pl.ds(fb, FROWS)])

    @pl.when(c == 1)
    def _():
        pltpu.sync_copy(p_hbm.at[pl.ds(fb, FROWS), pl.ds(96, 16)],
                        t1l.at[pl.ds(fb, FROWS)])
        pltpu.sync_copy(p_hbm.at[pl.ds(fb, FROWS), pl.ds(112, 16)],
                        t1r.at[pl.ds(fb, FROWS)])

    def _zr(e, carry):
        xl0[e, :] = jnp.zeros((16,), jnp.float32)
        return carry
    lax.fori_loop(0, 256, _zr, 0)
    zb = s * ZROWS
    for t in range(24):
        pltpu.sync_copy(xl0, accum.at[pl.ds(zb + t * 256, 256)])
    pltpu.sync_copy(xl0.at[pl.ds(0, ZROWS - 6144)],
                    accum.at[pl.ds(zb + 6144, ZROWS - 6144)])
    plsc.subcore_barrier()

    def conv(xl_tab, xr_tab):
        def fire_idx(chunk_i, q):
            rowbase = s * ROWS_PER_TILE + chunk_i * CHUNK_ROWS
            pltpu.async_copy(ei3.at[0, pl.ds(rowbase, CHUNK_ROWS)],
                             idxbufs[q][0], sem_is[q])
            pltpu.async_copy(ei3.at[1, pl.ds(rowbase, CHUNK_ROWS)],
                             idxbufs[q][1], sem_is[q])

        def wait_idx(q):
            pltpu.make_async_copy(ei3.at[0, pl.ds(0, CHUNK_ROWS)],
                                  idxbufs[q][0], sem_is[q]).wait()
            pltpu.make_async_copy(ei3.at[1, pl.ds(0, CHUNK_ROWS)],
                                  idxbufs[q][1], sem_is[q]).wait()

        def fire_gathers(q, r):
            sb, db = idxbufs[q]
            xlb, xrb = rowbufs[r]
            for j in range(CHUNK_ROWS):
                pltpu.async_copy(xl_tab.at[sb.at[j]],
                                 xlb.at[pl.ds(j * 128, 128)], sem_g)
                pltpu.async_copy(xr_tab.at[db.at[j]],
                                 xrb.at[pl.ds(j * 128, 128)], sem_g)

        def wait_gathers(r):
            xlb, xrb = rowbufs[r]
            for j in range(CHUNK_ROWS):
                pltpu.make_async_copy(xl_tab.at[sb0.at[j]],
                                      xlb.at[pl.ds(j * 128, 128)], sem_g).wait()
                pltpu.make_async_copy(xr_tab.at[db0.at[j]],
                                      xrb.at[pl.ds(j * 128, 128)], sem_g).wait()

        def one_group(xlb, xrb, base):
            rowi = base + lax.iota(jnp.int32, 16)
            l16a = jnp.zeros((16,), jnp.float32)
            l16b = jnp.zeros((16,), jnp.float32)
            acols = []
            for k in range(10):
                a = plsc.load_gather(xlb, [rowi, cols[k]])
                b = plsc.load_gather(xrb, [rowi, cols[k]])
                u = a + b
                m = jnp.where(u >= 0.0, u, 0.2 * u)
                if k % 2 == 0:
                    l16a = l16a + atts[k] * m
                else:
                    l16b = l16b + atts[k] * m
                acols.append(a)
            w16 = jnp.exp(l16a + l16b - cv)
            for k in range(10):
                plsc.store_scatter(xlb, [rowi, cols[k]], acols[k] * w16)
            plsc.store_scatter(xlb, [rowi, cols[10]], w16)

        def compute(r):
            xlb, xrb = rowbufs[r]

            def grp(g, carry2):
                one_group(xlb, xrb, g * 32)
                one_group(xlb, xrb, g * 32 + 16)
                return carry2
            lax.fori_loop(0, 8, grp, 0)

        def scatter(q, r):
            db = idxbufs[q][1]
            xlb = rowbufs[r][0]
            for j in range(CHUNK_ROWS):
                pltpu.sync_copy(xlb.at[pl.ds(j * 128, 128)],
                                accum.at[db.at[j]], add=True)

        rb0 = s * ROWS_PER_TILE
        pltpu.sync_copy(ei3.at[0, pl.ds(rb0, CHUNK_ROWS)], sb0)
        pltpu.sync_copy(ei3.at[1, pl.ds(rb0, CHUNK_ROWS)], db0)
        fire_gathers(0, 0)
        fire_idx(1, 1)

        def body(i4, carry):
            for t in range(4):
                i = i4 * 4 + t
                r = t % 2
                wait_gathers(r)

                @pl.when(i < CHUNKS - 1)
                def _():
                    wait_idx((t + 1) % 4)
                    fire_gathers((t + 1) % 4, 1 - r)

                @pl.when(i < CHUNKS - 2)
                def _():
                    fire_idx(i + 2, (t + 2) % 4)

                compute(r)
                scatter(t, r)
            return carry
        lax.fori_loop(0, CHUNKS // 4, body, 0)

    @pl.when(c == 0)
    def _():
        conv(t0l, t0r)

    @pl.when(c == 1)
    def _():
        conv(t1l, t1r)

    plsc.subcore_barrier()

    @pl.when(c == 0)
    def _():
        pltpu.sync_copy(accum.at[pl.ds(fb, FROWS)],
                        acc_out.at[pl.ds(fb, FROWS), pl.ds(0, 16)])

    @pl.when(c == 1)
    def _():
        pltpu.sync_copy(accum.at[pl.ds(fb, FROWS)],
                        acc_out.at[pl.ds(fb, FROWS), pl.ds(16, 16)])


def _gat_sc(ei3, p, att2, cv2):
    mesh = plsc.VectorSubcoreMesh(core_axis_name="c", subcore_axis_name="s")
    return pl.kernel(
        _gat_sc_body,
        out_type=jax.ShapeDtypeStruct((N, 128), jnp.float32),
        mesh=mesh,
        compiler_params=pltpu.CompilerParams(needs_layout_passes=False,
                                             use_tc_tiling_on_sc=False),
        scratch_types=(
            [pltpu.VMEM_SHARED((NPAD, 16), jnp.float32)]
            + [pltpu.HBM((NPAD, 16), jnp.float32)] * 4
            + [pltpu.VMEM((CHUNK_ROWS, 128), jnp.int32)] * 8
            + [pltpu.VMEM((256, 16), jnp.float32)] * 4
            + [pltpu.VMEM((16,), jnp.float32)] * 2
            + [pltpu.SemaphoreType.DMA] * 5
        ),
    )(ei3, p, att2, cv2)


def _fin_body(p_ref, acc_ref, eps_ref, bm_ref, bv_ref,
              zall_ref, qall_ref, lib_ref):
    p = p_ref[...]
    a = acc_ref[...]
    accm = a[:, 0:16]
    accv = a[:, 16:32]
    qgm = accm[:, 0:10] / (accm[:, 10:11] + 1e-16) + bm_ref[...]
    vlin = accv[:, 0:10] / (accv[:, 10:11] + 1e-16) + bv_ref[...]
    qgv = jnp.exp(vlin) + VAR_EPS
    z_gat = qgm + jnp.sqrt(qgv) * eps_ref[...]
    zall_ref[...] = jnp.concatenate([z_gat, p[:, 0:20]], axis=1)
    qall_ref[...] = jnp.concatenate([qgm, p[:, 20:40]], axis=1)
    lib_ref[...] = p[:, 40:41]


def _finalize(p, acc, eps_gat, gm_bias, gv_bias):
    full = lambda shape: pl.BlockSpec(shape, lambda i: (0, 0))
    blk = lambda shape: pl.BlockSpec(shape, lambda i: (i, 0))
    return pl.pallas_call(
        _fin_body,
        grid=(NB,),
        in_specs=[blk((BN, 128)), blk((BN, 128)),
                  blk((BN, 10)), full((1, 10)), full((1, 10))],
        out_specs=[blk((BN, 30)), blk((BN, 30)), blk((BN, 1))],
        out_shape=[jax.ShapeDtypeStruct((N, 30), jnp.float32),
                   jax.ShapeDtypeStruct((N, 30), jnp.float32),
                   jax.ShapeDtypeStruct((N, 1), jnp.float32)],
    )(p, acc, eps_gat, gm_bias, gv_bias)


def _pad16(v):
    return jnp.concatenate([v, jnp.zeros((6,), v.dtype)])


def kernel(x, batch_index, edge_index, W0, b0, W1, b1, Wm, bm, Wv, bv,
           gm_Wl, gm_bl, gm_Wr, gm_br, gm_att, gm_bias,
           gv_Wl, gv_bl, gv_Wr, gv_br, gv_att, gv_bias,
           eps_z, eps_gat):
    att2 = jnp.stack([_pad16(gm_att), _pad16(gv_att)])
    atta = jnp.abs(att2)
    p, bnd = _encoder(
        x, W0.T, b0.reshape(1, -1), W1.T, b1.reshape(1, -1),
        Wm.T, bm.reshape(1, -1), Wv.T, bv.reshape(1, -1),
        gm_Wl.T, gm_bl.reshape(1, -1), gm_Wr.T, gm_br.reshape(1, -1),
        gv_Wl.T, gv_bl.reshape(1, -1), gv_Wr.T, gv_br.reshape(1, -1),
        atta, eps_z)
    cm = bnd[0, 0] + bnd[0, 1]
    cvv = bnd[0, 2] + bnd[0, 3]
    cv2 = jnp.stack([jnp.full((16,), cm, jnp.float32),
                     jnp.full((16,), cvv, jnp.float32)])
    ei_pad = jnp.concatenate(
        [edge_index, jnp.full((2, EPAD), N, jnp.int32)], axis=1
    ).reshape(2, EROWS, 128)
    acc = _gat_sc(ei_pad, p, att2, cv2)
    z_all, qall_m, lib = _finalize(p, acc, eps_gat,
                                   gm_bias.reshape(1, -1),
                                   gv_bias.reshape(1, -1))
    return z_all, qall_m, lib


# speedup vs baseline: 1.4388x; 1.4388x over previous
import jax
import jax.numpy as jnp
from jax import lax
from jax.experimental import pallas as pl
from jax.experimental.pallas import tpu as pltpu
from jax.experimental.pallas import tpu_sc as plsc

N = 100000
E = 3200000
NB = 100
BN = N // NB
EPAD = 11264
EROWS = (E + EPAD) // 128
ROWS_PER_TILE = EROWS // 16
CHUNK_ROWS = 2
CHUNKS = ROWS_PER_TILE // CHUNK_ROWS
NPAD = 100096
ZROWS = NPAD // 16
FROWS = N // 16
VAR_EPS = 1e-4


def _enc_body(x_ref, w0_ref, b0_ref, w1_ref, b1_ref, wm_ref, bm_ref,
              wv_ref, bv_ref, wlm_ref, blm_ref, wrm_ref, brm_ref,
              wlv_ref, blv_ref, wrv_ref, brv_ref, atta_ref, eps_ref,
              p_ref, bnd_ref):
    i = pl.program_id(0)
    x = x_ref[...]
    lib = jnp.log(jnp.sum(x, axis=1, keepdims=True))
    xo = jnp.log1p(x)
    h = jax.nn.relu(jnp.dot(xo, w0_ref[...], preferred_element_type=jnp.float32) + b0_ref[...])
    h = jax.nn.relu(jnp.dot(h, w1_ref[...], preferred_element_type=jnp.float32) + b1_ref[...])
    qm = jnp.dot(h, wm_ref[...], preferred_element_type=jnp.float32) + bm_ref[...]
    qv = jnp.exp(jnp.dot(h, wv_ref[...], preferred_element_type=jnp.float32) + bv_ref[...]) + VAR_EPS
    z = qm + jnp.sqrt(qv) * eps_ref[...]
    qs = qm[:, 10:20]
    xlm = jnp.dot(qs, wlm_ref[...], preferred_element_type=jnp.float32) + blm_ref[...]
    xrm = jnp.dot(qs, wrm_ref[...], preferred_element_type=jnp.float32) + brm_ref[...]
    xlv = jnp.dot(qs, wlv_ref[...], preferred_element_type=jnp.float32) + blv_ref[...]
    xrv = jnp.dot(qs, wrv_ref[...], preferred_element_type=jnp.float32) + brv_ref[...]
    one = jnp.ones((BN, 1), jnp.float32)
    zero5 = jnp.zeros((BN, 5), jnp.float32)
    zero6 = jnp.zeros((BN, 6), jnp.float32)
    pad23 = jnp.zeros((BN, 23), jnp.float32)
    p_ref[...] = jnp.concatenate(
        [z, qm, lib, pad23,
         xlm, one, zero5, xrm, zero6,
         xlv, one, zero5, xrv, zero6], axis=1)
    am = atta_ref[0:1, 0:10]
    av = atta_ref[1:2, 0:10]
    pm = jnp.max(jnp.sum(jnp.abs(xlm) * am, axis=1))
    qmx = jnp.max(jnp.sum(jnp.abs(xrm) * am, axis=1))
    pv = jnp.max(jnp.sum(jnp.abs(xlv) * av, axis=1))
    qvx = jnp.max(jnp.sum(jnp.abs(xrv) * av, axis=1))
    lane = lax.broadcasted_iota(jnp.int32, (1, 128), 1)
    row = (jnp.where(lane == 0, pm, 0.0) + jnp.where(lane == 1, qmx, 0.0)
           + jnp.where(lane == 2, pv, 0.0) + jnp.where(lane == 3, qvx, 0.0))

    @pl.when(i == 0)
    def _():
        bnd_ref[...] = row

    @pl.when(i > 0)
    def _():
        bnd_ref[...] = jnp.maximum(bnd_ref[...], row)


def _encoder(x, w0t, b0, w1t, b1, wmt, bm, wvt, bv,
             wlmt, blm, wrmt, brm, wlvt, blv, wrvt, brv, atta, eps_z):
    full = lambda shape: pl.BlockSpec(shape, lambda i: (0, 0))
    blk = lambda shape: pl.BlockSpec(shape, lambda i: (i, 0))
    return pl.pallas_call(
        _enc_body,
        grid=(NB,),
        in_specs=[blk((BN, 128)), full((128, 128)), full((1, 128)),
                  full((128, 128)), full((1, 128)),
                  full((128, 20)), full((1, 20)), full((128, 20)), full((1, 20)),
                  full((10, 10)), full((1, 10)), full((10, 10)), full((1, 10)),
                  full((10, 10)), full((1, 10)), full((10, 10)), full((1, 10)),
                  full((2, 16)), blk((BN, 20))],
        out_specs=[blk((BN, 128)), full((1, 128))],
        out_shape=[jax.ShapeDtypeStruct((N, 128), jnp.float32),
                   jax.ShapeDtypeStruct((1, 128), jnp.float32)],
    )(x, w0t, b0, w1t, b1, wmt, bm, wvt, bv,
      wlmt, blm, wrmt, brm, wlvt, blv, wrvt, brv, atta, eps_z)


def _gat_sc_body(ei3, t0l, t0r, t1l, t1r, att2, cv2, acc_out,
                 accum,
                 sb0, db0, sb1, db1, sb2, db2, sb3, db3,
                 xl0, xr0, xl1, xr1, attb, cvb,
                 sem_g, sem_i0, sem_i1, sem_i2, sem_i3):
    c = lax.axis_index("c")
    s = lax.axis_index("s")
    idxbufs = [(sb0, db0), (sb1, db1), (sb2, db2), (sb3, db3)]
    rowbufs = [(xl0, xr0), (xl1, xr1)]
    sem_is = [sem_i0, sem_i1, sem_i2, sem_i3]

    pltpu.sync_copy(att2.at[c], attb)
    pltpu.sync_copy(cv2.at[c], cvb)
    cv = cvb[...]
    attv = attb[...]
    atts = [attv[k] for k in range(10)]
    cols = [jnp.full((16,), k, jnp.int32) for k in range(11)]

    fb = s * FROWS

    def _zr(e, carry):
        xl0[e, :] = jnp.zeros((16,), jnp.float32)
        return carry
    lax.fori_loop(0, 256, _zr, 0)
    zb = s * ZROWS
    for t in range(24):
        pltpu.sync_copy(xl0, accum.at[pl.ds(zb + t * 256, 256)])
    pltpu.sync_copy(xl0.at[pl.ds(0, ZROWS - 6144)],
                    accum.at[pl.ds(zb + 6144, ZROWS - 6144)])
    plsc.subcore_barrier()

    def conv(xl_tab, xr_tab):
        def fire_idx(chunk_i, q):
            rowbase = s * ROWS_PER_TILE + chunk_i * CHUNK_ROWS
            pltpu.async_copy(ei3.at[0, pl.ds(rowbase, CHUNK_ROWS)],
                             idxbufs[q][0], sem_is[q])
            pltpu.async_copy(ei3.at[1, pl.ds(rowbase, CHUNK_ROWS)],
                             idxbufs[q][1], sem_is[q])

        def wait_idx(q):
            pltpu.make_async_copy(ei3.at[0, pl.ds(0, CHUNK_ROWS)],
                                  idxbufs[q][0], sem_is[q]).wait()
            pltpu.make_async_copy(ei3.at[1, pl.ds(0, CHUNK_ROWS)],
                                  idxbufs[q][1], sem_is[q]).wait()

        def fire_gathers(q, r):
            sb, db = idxbufs[q]
            xlb, xrb = rowbufs[r]
            for j in range(CHUNK_ROWS):
                pltpu.async_copy(xl_tab.at[sb.at[j]],
                                 xlb.at[pl.ds(j * 128, 128)], sem_g)
                pltpu.async_copy(xr_tab.at[db.at[j]],
                                 xrb.at[pl.ds(j * 128, 128)], sem_g)

        def wait_gathers(r):
            xlb, xrb = rowbufs[r]
            for j in range(CHUNK_ROWS):
                pltpu.make_async_copy(xl_tab.at[sb0.at[j]],
                                      xlb.at[pl.ds(j * 128, 128)], sem_g).wait()
                pltpu.make_async_copy(xr_tab.at[db0.at[j]],
                                      xrb.at[pl.ds(j * 128, 128)], sem_g).wait()

        def one_group(xlb, xrb, base):
            rowi = base + lax.iota(jnp.int32, 16)
            l16a = jnp.zeros((16,), jnp.float32)
            l16b = jnp.zeros((16,), jnp.float32)
            acols = []
            for k in range(10):
                a = plsc.load_gather(xlb, [rowi, cols[k]])
                b = plsc.load_gather(xrb, [rowi, cols[k]])
                u = a + b
                m = jnp.where(u >= 0.0, u, 0.2 * u)
                if k % 2 == 0:
                    l16a = l16a + atts[k] * m
                else:
                    l16b = l16b + atts[k] * m
                acols.append(a)
            w16 = jnp.exp(l16a + l16b - cv)
            for k in range(10):
                plsc.store_scatter(xlb, [rowi, cols[k]], acols[k] * w16)
            plsc.store_scatter(xlb, [rowi, cols[10]], w16)

        def compute(r):
            xlb, xrb = rowbufs[r]

            def grp(g, carry2):
                one_group(xlb, xrb, g * 32)
                one_group(xlb, xrb, g * 32 + 16)
                return carry2
            lax.fori_loop(0, 8, grp, 0)

        def scatter(q, r):
            db = idxbufs[q][1]
            xlb = rowbufs[r][0]
            for j in range(CHUNK_ROWS):
                pltpu.sync_copy(xlb.at[pl.ds(j * 128, 128)],
                                accum.at[db.at[j]], add=True)

        rb0 = s * ROWS_PER_TILE
        pltpu.sync_copy(ei3.at[0, pl.ds(rb0, CHUNK_ROWS)], sb0)
        pltpu.sync_copy(ei3.at[1, pl.ds(rb0, CHUNK_ROWS)], db0)
        fire_gathers(0, 0)
        fire_idx(1, 1)

        def body(i4, carry):
            for t in range(4):
                i = i4 * 4 + t
                r = t % 2
                wait_gathers(r)

                @pl.when(i < CHUNKS - 1)
                def _():
                    wait_idx((t + 1) % 4)
                    fire_gathers((t + 1) % 4, 1 - r)

                @pl.when(i < CHUNKS - 2)
                def _():
                    fire_idx(i + 2, (t + 2) % 4)

                compute(r)
                scatter(t, r)
            return carry
        lax.fori_loop(0, CHUNKS // 4, body, 0)

    @pl.when(c == 0)
    def _():
        conv(t0l, t0r)

    @pl.when(c == 1)
    def _():
        conv(t1l, t1r)

    plsc.subcore_barrier()

    @pl.when(c == 0)
    def _():
        pltpu.sync_copy(accum.at[pl.ds(fb, FROWS)],
                        acc_out.at[pl.ds(fb, FROWS), pl.ds(0, 16)])

    @pl.when(c == 1)
    def _():
        pltpu.sync_copy(accum.at[pl.ds(fb, FROWS)],
                        acc_out.at[pl.ds(fb, FROWS), pl.ds(16, 16)])


def _gat_sc(ei3, t0l, t0r, t1l, t1r, att2, cv2):
    mesh = plsc.VectorSubcoreMesh(core_axis_name="c", subcore_axis_name="s")
    return pl.kernel(
        _gat_sc_body,
        out_type=jax.ShapeDtypeStruct((N, 128), jnp.float32),
        mesh=mesh,
        compiler_params=pltpu.CompilerParams(needs_layout_passes=False,
                                             use_tc_tiling_on_sc=False),
        scratch_types=(
            [pltpu.VMEM_SHARED((NPAD, 16), jnp.float32)]
            + [pltpu.VMEM((CHUNK_ROWS, 128), jnp.int32)] * 8
            + [pltpu.VMEM((256, 16), jnp.float32)] * 4
            + [pltpu.VMEM((16,), jnp.float32)] * 2
            + [pltpu.SemaphoreType.DMA] * 5
        ),
    )(ei3, t0l, t0r, t1l, t1r, att2, cv2)


def _fin_body(p_ref, acc_ref, eps_ref, bm_ref, bv_ref,
              zall_ref, qall_ref, lib_ref):
    p = p_ref[...]
    a = acc_ref[...]
    accm = a[:, 0:16]
    accv = a[:, 16:32]
    qgm = accm[:, 0:10] / (accm[:, 10:11] + 1e-16) + bm_ref[...]
    vlin = accv[:, 0:10] / (accv[:, 10:11] + 1e-16) + bv_ref[...]
    qgv = jnp.exp(vlin) + VAR_EPS
    z_gat = qgm + jnp.sqrt(qgv) * eps_ref[...]
    zall_ref[...] = jnp.concatenate([z_gat, p[:, 0:20]], axis=1)
    qall_ref[...] = jnp.concatenate([qgm, p[:, 20:40]], axis=1)
    lib_ref[...] = p[:, 40:41]


def _finalize(p, acc, eps_gat, gm_bias, gv_bias):
    full = lambda shape: pl.BlockSpec(shape, lambda i: (0, 0))
    blk = lambda shape: pl.BlockSpec(shape, lambda i: (i, 0))
    return pl.pallas_call(
        _fin_body,
        grid=(NB,),
        in_specs=[blk((BN, 128)), blk((BN, 128)),
                  blk((BN, 10)), full((1, 10)), full((1, 10))],
        out_specs=[blk((BN, 30)), blk((BN, 30)), blk((BN, 1))],
        out_shape=[jax.ShapeDtypeStruct((N, 30), jnp.float32),
                   jax.ShapeDtypeStruct((N, 30), jnp.float32),
                   jax.ShapeDtypeStruct((N, 1), jnp.float32)],
    )(p, acc, eps_gat, gm_bias, gv_bias)


def _pad16(v):
    return jnp.concatenate([v, jnp.zeros((6,), v.dtype)])


def kernel(x, batch_index, edge_index, W0, b0, W1, b1, Wm, bm, Wv, bv,
           gm_Wl, gm_bl, gm_Wr, gm_br, gm_att, gm_bias,
           gv_Wl, gv_bl, gv_Wr, gv_br, gv_att, gv_bias,
           eps_z, eps_gat):
    att2 = jnp.stack([_pad16(gm_att), _pad16(gv_att)])
    atta = jnp.abs(att2)
    p, bnd = _encoder(
        x, W0.T, b0.reshape(1, -1), W1.T, b1.reshape(1, -1),
        Wm.T, bm.reshape(1, -1), Wv.T, bv.reshape(1, -1),
        gm_Wl.T, gm_bl.reshape(1, -1), gm_Wr.T, gm_br.reshape(1, -1),
        gv_Wl.T, gv_bl.reshape(1, -1), gv_Wr.T, gv_br.reshape(1, -1),
        atta, eps_z)
    cm = bnd[0, 0] + bnd[0, 1]
    cvv = bnd[0, 2] + bnd[0, 3]
    cv2 = jnp.stack([jnp.full((16,), cm, jnp.float32),
                     jnp.full((16,), cvv, jnp.float32)])
    ei_pad = jnp.concatenate(
        [edge_index, jnp.full((2, EPAD), N, jnp.int32)], axis=1
    ).reshape(2, EROWS, 128)
    padrows = jnp.zeros((NPAD - N, 16), jnp.float32)
    acc = _gat_sc(ei_pad,
                  jnp.concatenate([p[:, 64:80], padrows]),
                  jnp.concatenate([p[:, 80:96], padrows]),
                  jnp.concatenate([p[:, 96:112], padrows]),
                  jnp.concatenate([p[:, 112:128], padrows]),
                  att2, cv2)
    z_all, qall_m, lib = _finalize(p, acc, eps_gat,
                                   gm_bias.reshape(1, -1),
                                   gv_bias.reshape(1, -1))
    return z_all, qall_m, lib


# speedup vs baseline: 1.6299x; 1.1328x over previous
import jax
import jax.numpy as jnp
from jax import lax
from jax.experimental import pallas as pl
from jax.experimental.pallas import tpu as pltpu
from jax.experimental.pallas import tpu_sc as plsc

N = 100000
E = 3200000
NB = 100
BN = N // NB
EPAD = 11264
EROWS = (E + EPAD) // 128
ROWS_PER_TILE = EROWS // 16
CHUNK_ROWS = 2
CHUNKS = ROWS_PER_TILE // CHUNK_ROWS
NPAD = 100096
ZROWS = NPAD // 16
FROWS = N // 16
VAR_EPS = 1e-4


def _enc_body(x_ref, w0_ref, b0_ref, w1_ref, b1_ref, wbig_ref, bbig_ref,
              wvw_ref, bvw_ref, att4_ref, eps_ref,
              p_ref, bnd_ref):
    i = pl.program_id(0)
    x = x_ref[...]
    lib = jnp.log(jnp.sum(x, axis=1, keepdims=True))
    xo = jnp.log1p(x)
    h = jax.nn.relu(jnp.dot(xo, w0_ref[...], preferred_element_type=jnp.float32) + b0_ref[...])
    h = jax.nn.relu(jnp.dot(h, w1_ref[...], preferred_element_type=jnp.float32) + b1_ref[...])
    p0 = jnp.dot(h, wbig_ref[...], preferred_element_type=jnp.float32) + bbig_ref[...]
    qvw = jnp.exp(jnp.dot(h, wvw_ref[...], preferred_element_type=jnp.float32)
                  + bvw_ref[...]) + VAR_EPS
    epsw = jnp.concatenate(
        [eps_ref[...], jnp.zeros((BN, 108), jnp.float32)], axis=1)
    lane = lax.broadcasted_iota(jnp.int32, (1, 128), 1)
    p = p0 + jnp.sqrt(qvw) * epsw + jnp.where(lane == 40, lib, 0.0)
    p_ref[...] = p
    bndmat = jnp.dot(jnp.abs(p), att4_ref[...],
                     preferred_element_type=jnp.float32)
    m0 = jnp.max(bndmat[:, 0])
    m1 = jnp.max(bndmat[:, 1])
    m2 = jnp.max(bndmat[:, 2])
    m3 = jnp.max(bndmat[:, 3])
    row = (jnp.where(lane == 0, m0, 0.0) + jnp.where(lane == 1, m1, 0.0)
           + jnp.where(lane == 2, m2, 0.0) + jnp.where(lane == 3, m3, 0.0))

    @pl.when(i == 0)
    def _():
        bnd_ref[...] = row

    @pl.when(i > 0)
    def _():
        bnd_ref[...] = jnp.maximum(bnd_ref[...], row)


def _encoder(x, w0t, b0, w1t, b1, wbig, bbig, wvw, bvw, att4, eps_z):
    full = lambda shape: pl.BlockSpec(shape, lambda i: (0, 0))
    blk = lambda shape: pl.BlockSpec(shape, lambda i: (i, 0))
    return pl.pallas_call(
        _enc_body,
        grid=(NB,),
        in_specs=[blk((BN, 128)), full((128, 128)), full((1, 128)),
                  full((128, 128)), full((1, 128)),
                  full((128, 128)), full((1, 128)),
                  full((128, 128)), full((1, 128)),
                  full((128, 4)), blk((BN, 20))],
        out_specs=[blk((BN, 128)), full((1, 128))],
        out_shape=[jax.ShapeDtypeStruct((N, 128), jnp.float32),
                   jax.ShapeDtypeStruct((1, 128), jnp.float32)],
    )(x, w0t, b0, w1t, b1, wbig, bbig, wvw, bvw, att4, eps_z)


def _gat_sc_body(ei3, t0l, t0r, t1l, t1r, att2, cv2, acc_out,
                 accum,
                 sb0, db0, sb1, db1, sb2, db2, sb3, db3,
                 xl0, xr0, xl1, xr1, attb, cvb,
                 sem_g, sem_i0, sem_i1, sem_i2, sem_i3):
    c = lax.axis_index("c")
    s = lax.axis_index("s")
    idxbufs = [(sb0, db0), (sb1, db1), (sb2, db2), (sb3, db3)]
    rowbufs = [(xl0, xr0), (xl1, xr1)]
    sem_is = [sem_i0, sem_i1, sem_i2, sem_i3]

    pltpu.sync_copy(att2.at[c], attb)
    pltpu.sync_copy(cv2.at[c], cvb)
    cv = cvb[...]
    attv = attb[...]
    atts = [attv[k] for k in range(10)]
    cols = [jnp.full((16,), k, jnp.int32) for k in range(11)]

    fb = s * FROWS

    def _zr(e, carry):
        xl0[e, :] = jnp.zeros((16,), jnp.float32)
        return carry
    lax.fori_loop(0, 256, _zr, 0)
    zb = s * ZROWS
    for t in range(24):
        pltpu.sync_copy(xl0, accum.at[pl.ds(zb + t * 256, 256)])
    pltpu.sync_copy(xl0.at[pl.ds(0, ZROWS - 6144)],
                    accum.at[pl.ds(zb + 6144, ZROWS - 6144)])
    plsc.subcore_barrier()

    def conv(xl_tab, xr_tab):
        def fire_idx(chunk_i, q):
            rowbase = s * ROWS_PER_TILE + chunk_i * CHUNK_ROWS
            pltpu.async_copy(ei3.at[0, pl.ds(rowbase, CHUNK_ROWS)],
                             idxbufs[q][0], sem_is[q])
            pltpu.async_copy(ei3.at[1, pl.ds(rowbase, CHUNK_ROWS)],
                             idxbufs[q][1], sem_is[q])

        def wait_idx(q):
            pltpu.make_async_copy(ei3.at[0, pl.ds(0, CHUNK_ROWS)],
                                  idxbufs[q][0], sem_is[q]).wait()
            pltpu.make_async_copy(ei3.at[1, pl.ds(0, CHUNK_ROWS)],
                                  idxbufs[q][1], sem_is[q]).wait()

        def fire_gathers(q, r):
            sb, db = idxbufs[q]
            xlb, xrb = rowbufs[r]
            for j in range(CHUNK_ROWS):
                pltpu.async_copy(xl_tab.at[sb.at[j]],
                                 xlb.at[pl.ds(j * 128, 128)], sem_g)
                pltpu.async_copy(xr_tab.at[db.at[j]],
                                 xrb.at[pl.ds(j * 128, 128)], sem_g)

        def wait_gathers(r):
            xlb, xrb = rowbufs[r]
            for j in range(CHUNK_ROWS):
                pltpu.make_async_copy(xl_tab.at[sb0.at[j]],
                                      xlb.at[pl.ds(j * 128, 128)], sem_g).wait()
                pltpu.make_async_copy(xr_tab.at[db0.at[j]],
                                      xrb.at[pl.ds(j * 128, 128)], sem_g).wait()

        def one_group(xlb, xrb, base):
            rowi = base + lax.iota(jnp.int32, 16)
            l16a = jnp.zeros((16,), jnp.float32)
            l16b = jnp.zeros((16,), jnp.float32)
            acols = []
            for k in range(10):
                a = plsc.load_gather(xlb, [rowi, cols[k]])
                b = plsc.load_gather(xrb, [rowi, cols[k]])
                u = a + b
                m = jnp.where(u >= 0.0, u, 0.2 * u)
                if k % 2 == 0:
                    l16a = l16a + atts[k] * m
                else:
                    l16b = l16b + atts[k] * m
                acols.append(a)
            w16 = jnp.exp(l16a + l16b - cv)
            for k in range(10):
                plsc.store_scatter(xlb, [rowi, cols[k]], acols[k] * w16)
            plsc.store_scatter(xlb, [rowi, cols[10]], w16)

        def compute(r):
            xlb, xrb = rowbufs[r]

            def grp(g, carry2):
                one_group(xlb, xrb, g * 32)
                one_group(xlb, xrb, g * 32 + 16)
                return carry2
            lax.fori_loop(0, 8, grp, 0)

        def scatter(q, r):
            db = idxbufs[q][1]
            xlb = rowbufs[r][0]
            for j in range(CHUNK_ROWS):
                pltpu.sync_copy(xlb.at[pl.ds(j * 128, 128)],
                                accum.at[db.at[j]], add=True)

        rb0 = s * ROWS_PER_TILE
        pltpu.sync_copy(ei3.at[0, pl.ds(rb0, CHUNK_ROWS)], sb0)
        pltpu.sync_copy(ei3.at[1, pl.ds(rb0, CHUNK_ROWS)], db0)
        fire_gathers(0, 0)
        fire_idx(1, 1)

        def body(i4, carry):
            for t in range(4):
                i = i4 * 4 + t
                r = t % 2
                wait_gathers(r)

                @pl.when(i < CHUNKS - 1)
                def _():
                    wait_idx((t + 1) % 4)
                    fire_gathers((t + 1) % 4, 1 - r)

                @pl.when(i < CHUNKS - 2)
                def _():
                    fire_idx(i + 2, (t + 2) % 4)

                compute(r)
                scatter(t, r)
            return carry
        lax.fori_loop(0, CHUNKS // 4, body, 0)

    @pl.when(c == 0)
    def _():
        conv(t0l, t0r)

    @pl.when(c == 1)
    def _():
        conv(t1l, t1r)

    plsc.subcore_barrier()

    @pl.when(c == 0)
    def _():
        pltpu.sync_copy(accum.at[pl.ds(fb, FROWS)],
                        acc_out.at[pl.ds(fb, FROWS), pl.ds(0, 16)])

    @pl.when(c == 1)
    def _():
        pltpu.sync_copy(accum.at[pl.ds(fb, FROWS)],
                        acc_out.at[pl.ds(fb, FROWS), pl.ds(16, 16)])


def _gat_sc(ei3, t0l, t0r, t1l, t1r, att2, cv2):
    mesh = plsc.VectorSubcoreMesh(core_axis_name="c", subcore_axis_name="s")
    return pl.kernel(
        _gat_sc_body,
        out_type=jax.ShapeDtypeStruct((N, 128), jnp.float32),
        mesh=mesh,
        compiler_params=pltpu.CompilerParams(needs_layout_passes=False,
                                             use_tc_tiling_on_sc=False),
        scratch_types=(
            [pltpu.VMEM_SHARED((NPAD, 16), jnp.float32)]
            + [pltpu.VMEM((CHUNK_ROWS, 128), jnp.int32)] * 8
            + [pltpu.VMEM((256, 16), jnp.float32)] * 4
            + [pltpu.VMEM((16,), jnp.float32)] * 2
            + [pltpu.SemaphoreType.DMA] * 5
        ),
    )(ei3, t0l, t0r, t1l, t1r, att2, cv2)


def _fin_body(p_ref, acc_ref, eps_ref, bm_ref, bv_ref,
              zall_ref, qall_ref, lib_ref):
    p = p_ref[...]
    a = acc_ref[...]
    accm = a[:, 0:16]
    accv = a[:, 16:32]
    qgm = accm[:, 0:10] / (accm[:, 10:11] + 1e-16) + bm_ref[...]
    vlin = accv[:, 0:10] / (accv[:, 10:11] + 1e-16) + bv_ref[...]
    qgv = jnp.exp(vlin) + VAR_EPS
    z_gat = qgm + jnp.sqrt(qgv) * eps_ref[...]
    zall_ref[...] = jnp.concatenate([z_gat, p[:, 0:20]], axis=1)
    qall_ref[...] = jnp.concatenate([qgm, p[:, 20:40]], axis=1)
    lib_ref[...] = p[:, 40:41]


def _finalize(p, acc, eps_gat, gm_bias, gv_bias):
    full = lambda shape: pl.BlockSpec(shape, lambda i: (0, 0))
    blk = lambda shape: pl.BlockSpec(shape, lambda i: (i, 0))
    return pl.pallas_call(
        _fin_body,
        grid=(NB,),
        in_specs=[blk((BN, 128)), blk((BN, 128)),
                  blk((BN, 10)), full((1, 10)), full((1, 10))],
        out_specs=[blk((BN, 30)), blk((BN, 30)), blk((BN, 1))],
        out_shape=[jax.ShapeDtypeStruct((N, 30), jnp.float32),
                   jax.ShapeDtypeStruct((N, 30), jnp.float32),
                   jax.ShapeDtypeStruct((N, 1), jnp.float32)],
    )(p, acc, eps_gat, gm_bias, gv_bias)


def _pad16(v):
    return jnp.concatenate([v, jnp.zeros((6,), v.dtype)])


def kernel(x, batch_index, edge_index, W0, b0, W1, b1, Wm, bm, Wv, bv,
           gm_Wl, gm_bl, gm_Wr, gm_br, gm_att, gm_bias,
           gv_Wl, gv_bl, gv_Wr, gv_br, gv_att, gv_bias,
           eps_z, eps_gat):
    att2 = jnp.stack([_pad16(gm_att), _pad16(gv_att)])
    wmt = Wm.T
    wm2t = wmt[:, 10:20]
    bm2 = bm[10:20]
    z24 = jnp.zeros((128, 24), jnp.float32)
    z6 = jnp.zeros((128, 6), jnp.float32)
    wbig = jnp.concatenate(
        [wmt, wmt, z24,
         wm2t @ gm_Wl.T, z6, wm2t @ gm_Wr.T, z6,
         wm2t @ gv_Wl.T, z6, wm2t @ gv_Wr.T, z6], axis=1)
    one1 = jnp.ones((1,), jnp.float32)
    z5 = jnp.zeros((5,), jnp.float32)
    zb6 = jnp.zeros((6,), jnp.float32)
    bbig = jnp.concatenate(
        [bm, bm, jnp.zeros((24,), jnp.float32),
         bm2 @ gm_Wl.T + gm_bl, one1, z5, bm2 @ gm_Wr.T + gm_br, zb6,
         bm2 @ gv_Wl.T + gv_bl, one1, z5, bm2 @ gv_Wr.T + gv_br, zb6]
    ).reshape(1, 128)
    wvw = jnp.concatenate([Wv.T, jnp.zeros((128, 108), jnp.float32)], axis=1)
    bvw = jnp.concatenate([bv, jnp.full((108,), -30.0, jnp.float32)]
                          ).reshape(1, 128)
    am = jnp.abs(gm_att)
    av = jnp.abs(gv_att)
    att4 = jnp.zeros((128, 4), jnp.float32)
    att4 = att4.at[64:74, 0].set(am).at[80:90, 1].set(am)
    att4 = att4.at[96:106, 2].set(av).at[112:122, 3].set(av)
    p, bnd = _encoder(
        x, W0.T, b0.reshape(1, -1), W1.T, b1.reshape(1, -1),
        wbig, bbig, wvw, bvw, att4, eps_z)
    cm = bnd[0, 0] + bnd[0, 1]
    cvv = bnd[0, 2] + bnd[0, 3]
    cv2 = jnp.stack([jnp.full((16,), cm, jnp.float32),
                     jnp.full((16,), cvv, jnp.float32)])
    ei_pad = jnp.concatenate(
        [edge_index, jnp.full((2, EPAD), N, jnp.int32)], axis=1
    ).reshape(2, EROWS, 128)
    padrows = jnp.zeros((NPAD - N, 16), jnp.float32)
    acc = _gat_sc(ei_pad,
                  jnp.concatenate([p[:, 64:80], padrows]),
                  jnp.concatenate([p[:, 80:96], padrows]),
                  jnp.concatenate([p[:, 96:112], padrows]),
                  jnp.concatenate([p[:, 112:128], padrows]),
                  att2, cv2)
    z_all, qall_m, lib = _finalize(p, acc, eps_gat,
                                   gm_bias.reshape(1, -1),
                                   gv_bias.reshape(1, -1))
    return z_all, qall_m, lib
